# baseline (speedup 1.0000x reference)
import functools

import jax
import jax.numpy as jnp
from jax import lax
from jax.experimental import pallas as pl
from jax.experimental.pallas import tpu as pltpu
from jax.experimental.pallas import tpu_sc as plsc


def _route_gather_body(E, T, b_per_w,
                       x_hbm, x0_hbm, tid_hbm,
                       pos_hbm, sidx_hbm, offs_hbm,
                       xs_hbm, xs0_hbm,
                       tid_v, r_v, pos_v, sidx_v, offs_v,
                       shared_sidx, idx_v, rows_v, rows0_v, sem, sem0):
    cid = lax.axis_index("c")
    sid = lax.axis_index("s")

    @pl.when(sid == 0)
    def _():
        pltpu.sync_copy(tid_hbm, tid_v)
        iota16 = lax.iota(jnp.int32, 16)
        nchunks = T // 16

        def hash_count(i, counts):
            t = tid_v[pl.ds(i * 16, 16)]
            h = t.astype(jnp.uint32)
            h = h ^ (h >> 16)
            h = h * jnp.uint32(2246822507)
            h = h ^ (h >> 13)
            h = h * jnp.uint32(3266489909)
            h = h ^ (h >> 16)
            r = (h % jnp.uint32(E)).astype(jnp.int32)
            r_v[pl.ds(i * 16, 16)] = r
            for e in range(E):
                c = jnp.sum((r == e).astype(jnp.int32))
                counts = counts + jnp.where(iota16 == e, c, 0)
            return counts

        counts = lax.fori_loop(0, nchunks, hash_count,
                               jnp.zeros((16,), jnp.int32))
        offs = plsc.cumsum(counts) - counts
        offs_v[pl.ds(0, 16)] = offs

        def place(i, running):
            r = r_v[pl.ds(i * 16, 16)]
            base = jnp.zeros((16,), jnp.int32)
            newrun = running
            for e in range(E):
                m = r == e
                mi = m.astype(jnp.int32)
                pre = plsc.cumsum(mi)
                run_e = jnp.sum(jnp.where(iota16 == e, running, 0))
                base = jnp.where(m, run_e + pre - 1, base)
                newrun = newrun + jnp.where(iota16 == e, jnp.sum(mi), 0)
            pos_v[pl.ds(i * 16, 16)] = base
            tok = lax.iota(jnp.int32, 16) + i * 16
            plsc.store_scatter(sidx_v, [base], tok)
            return newrun

        lax.fori_loop(0, nchunks, place, offs)
        pltpu.sync_copy(sidx_v, shared_sidx)

    @pl.when((sid == 0) & (cid == 0))
    def _():
        pltpu.sync_copy(pos_v, pos_hbm)
        pltpu.sync_copy(sidx_v, sidx_hbm)
        pltpu.sync_copy(offs_v, offs_hbm)

    plsc.subcore_barrier()

    wid = sid * 2 + cid
    base = wid * b_per_w
    pltpu.sync_copy(shared_sidx.at[pl.ds(base, b_per_w)], idx_v)
    cp = pltpu.async_copy(x_hbm.at[idx_v], rows_v, sem)
    cp0 = pltpu.async_copy(x0_hbm.at[idx_v], rows0_v, sem0)
    cp.wait()
    cp0.wait()
    pltpu.sync_copy(rows_v, xs_hbm.at[pl.ds(base, b_per_w)])
    pltpu.sync_copy(rows0_v, xs0_hbm.at[pl.ds(base, b_per_w)])


def _make_route_gather(E, T, D):
    b_per_w = T // 32
    mesh = plsc.VectorSubcoreMesh(core_axis_name="c", subcore_axis_name="s")
    i32 = jnp.int32
    f32 = jnp.float32
    return functools.partial(
        pl.kernel,
        out_type=(jax.ShapeDtypeStruct((T,), i32),
                  jax.ShapeDtypeStruct((T,), i32),
                  jax.ShapeDtypeStruct((16,), i32),
                  jax.ShapeDtypeStruct((T, D), f32),
                  jax.ShapeDtypeStruct((T, D), f32)),
        mesh=mesh,
        scratch_types=[pltpu.VMEM((T,), i32),
                       pltpu.VMEM((T,), i32),
                       pltpu.VMEM((T,), i32),
                       pltpu.VMEM((T,), i32),
                       pltpu.VMEM((16,), i32),
                       pltpu.VMEM_SHARED((T,), i32),
                       pltpu.VMEM((b_per_w,), i32),
                       pltpu.VMEM((b_per_w, D), f32),
                       pltpu.VMEM((b_per_w, D), f32),
                       pltpu.SemaphoreType.DMA,
                       pltpu.SemaphoreType.DMA],
        compiler_params=pltpu.CompilerParams(needs_layout_passes=False),
    )(functools.partial(_route_gather_body, E, T, b_per_w))


def _gather1_body(b_per_w, x_hbm, idx_hbm, xs_hbm, idx_v, rows_v, sem):
    wid = lax.axis_index("s") * 2 + lax.axis_index("c")
    base = wid * b_per_w
    pltpu.sync_copy(idx_hbm.at[pl.ds(base, b_per_w)], idx_v)
    pltpu.async_copy(x_hbm.at[idx_v], rows_v, sem).wait()
    pltpu.sync_copy(rows_v, xs_hbm.at[pl.ds(base, b_per_w)])


def _make_gather1(T, D):
    b_per_w = T // 32
    mesh = plsc.VectorSubcoreMesh(core_axis_name="c", subcore_axis_name="s")
    return functools.partial(
        pl.kernel,
        out_type=jax.ShapeDtypeStruct((T, D), jnp.float32),
        mesh=mesh,
        scratch_types=[pltpu.VMEM((b_per_w,), jnp.int32),
                       pltpu.VMEM((b_per_w, D), jnp.float32),
                       pltpu.SemaphoreType.DMA],
    )(functools.partial(_gather1_body, b_per_w))


_BLK = 512


def _ssm_kernel(xs_ref, xs0_ref, offs_ref, rm_ref,
                wi_ref, wsi_ref, wso_ref, wo_ref, dp_ref, ssc_ref, msc_ref,
                w1_ref, w2_ref, ys_ref, hcar_ref):
    f32 = jnp.float32
    BLK = _BLK
    S = wi_ref.shape[2]
    H = wsi_ref.shape[2]
    D = wo_ref.shape[2]
    E = wi_ref.shape[0]
    pid = pl.program_id(0)

    @pl.when(pid == 0)
    def _():
        hcar_ref[...] = jnp.zeros_like(hcar_ref)

    xm = rm_ref[0:1, :] * xs_ref[...] + rm_ref[1:2, :] * xs0_ref[...]
    xn = xm * lax.rsqrt(jnp.mean(xm * xm, axis=-1, keepdims=True) + 1e-6)
    xnb = xn.astype(jnp.bfloat16)

    t0 = pid * BLK
    t_hi = t0 + BLK - 1
    ti = lax.broadcasted_iota(jnp.int32, (BLK, 1), 0) + t0
    e_lo = jnp.int32(0)
    e_hi = jnp.int32(0)
    for e in range(1, E):
        off_e = offs_ref[0, e]
        e_lo = jnp.where(off_e <= t0, jnp.int32(e), e_lo)
        e_hi = jnp.where(off_e <= t_hi, jnp.int32(e), e_hi)

    def _mask(e):
        lo = offs_ref[0, e]
        hi = offs_ref[0, e + 1]
        return ((ti >= lo) & (ti < hi)).astype(f32)

    def body1(e, carry):
        U, SELR = carry
        m = _mask(e)
        wi = wi_ref[pl.ds(e, 1)][0]
        wsi = wsi_ref[pl.ds(e, 1)][0]
        U = U + m * jnp.dot(xnb, wi, preferred_element_type=f32)
        SELR = SELR + m * jnp.dot(xnb, wsi, preferred_element_type=f32)
        return U, SELR

    U, SELR = lax.fori_loop(e_lo, e_hi + 1, body1,
                            (jnp.zeros((BLK, S), f32),
                             jnp.zeros((BLK, H), f32)))
    selb = (SELR * jax.nn.sigmoid(SELR)).astype(jnp.bfloat16)

    def body2(e, carry):
        SO, DP = carry
        m = _mask(e)
        wso = wso_ref[pl.ds(e, 1)][0]
        SO = SO + m * jnp.dot(selb, wso, preferred_element_type=f32)
        DP = DP + m * dp_ref[pl.ds(e, 1), :]
        return SO, DP

    SO, DP = lax.fori_loop(e_lo, e_hi + 1, body2,
                           (jnp.zeros((BLK, 4 * S), f32),
                            jnp.zeros((BLK, S), f32)))

    a = jax.nn.sigmoid(SO[:, 0:S])
    b = jnp.tanh(SO[:, S:2 * S])
    c = jnp.tanh(SO[:, 2 * S:3 * S])
    dg = jax.nn.sigmoid(SO[:, 3 * S:4 * S])

    amult = jnp.ones((BLK, 1), f32)
    for e in range(E):
        amult = jnp.where(ti == offs_ref[0, e], 0.0, amult)
    A = a * amult
    Hs = b * U
    ri = lax.broadcasted_iota(jnp.int32, (BLK, 1), 0)
    d = 1
    while d < BLK:
        h_sh = jnp.where(ri >= d, pltpu.roll(Hs, d, axis=0), 0.0)
        a_sh = jnp.where(ri >= d, pltpu.roll(A, d, axis=0), 1.0)
        Hs = Hs + A * h_sh
        A = A * a_sh
        d *= 2
    h = Hs + A * hcar_ref[0:1, :]
    hcar_ref[0:1, :] = h[BLK - 1:BLK, :]

    Y = c * h + DP * dg * U
    yb = Y.astype(jnp.bfloat16)

    def body3(e, OUT):
        m = _mask(e)
        wo = wo_ref[pl.ds(e, 1)][0]
        return OUT + m * jnp.dot(yb, wo, preferred_element_type=f32)

    ssm_out = lax.fori_loop(e_lo, e_hi + 1, body3, jnp.zeros((BLK, D), f32))

    xm2 = xm + ssc_ref[...] * ssm_out
    xn2 = xm2 * lax.rsqrt(jnp.mean(xm2 * xm2, axis=-1, keepdims=True) + 1e-6)
    hmid = jnp.dot(xn2.astype(jnp.bfloat16), w1_ref[...],
                   preferred_element_type=f32)
    hact = jnp.maximum(hmid, 0.0)
    hact = hact * hact
    mlp = jnp.dot(hact.astype(jnp.bfloat16), w2_ref[...],
                  preferred_element_type=f32)
    ys_ref[...] = xm2 + msc_ref[...] * mlp


def _run_ssm(xs, xs0, offs, resid_mix, wi, wsi, wso, wo, dp,
             ssc, msc, w1, w2):
    T, D = xs.shape
    E, _, S = wi.shape
    H = wsi.shape[2]
    F = w1.shape[1]
    nblk = T // _BLK
    full = lambda *shape: pl.BlockSpec(shape, lambda i: (0,) * len(shape))
    return pl.pallas_call(
        _ssm_kernel,
        grid=(nblk,),
        in_specs=[
            pl.BlockSpec((_BLK, D), lambda i: (i, 0)),
            pl.BlockSpec((_BLK, D), lambda i: (i, 0)),
            pl.BlockSpec((1, 16), lambda i: (0, 0),
                         memory_space=pltpu.SMEM),
            full(2, D),
            full(E, D, S),
            full(E, D, H),
            full(E, H, 4 * S),
            full(E, S, D),
            full(E, S),
            full(1, D),
            full(1, D),
            full(D, F),
            full(F, D),
        ],
        out_specs=pl.BlockSpec((_BLK, D), lambda i: (i, 0)),
        out_shape=jax.ShapeDtypeStruct((T, D), jnp.float32),
        scratch_shapes=[pltpu.VMEM((8, S), jnp.float32)],
    )(xs, xs0, offs.reshape(1, 16), resid_mix, wi, wsi, wso, wo, dp,
      ssc, msc, w1, w2)


def _cast_kernel(wi_ref, wsi_ref, wso_ref, wo_ref, w1_ref, w2_ref,
                 owi_ref, owsi_ref, owso_ref, owo_ref, ow1_ref, ow2_ref):
    bf16 = jnp.bfloat16
    owi_ref[...] = wi_ref[...].astype(bf16)
    owsi_ref[...] = wsi_ref[...].astype(bf16)
    owso_ref[...] = wso_ref[...].astype(bf16)
    owo_ref[...] = wo_ref[...].astype(bf16)
    ow1_ref[...] = w1_ref[...].astype(bf16)
    ow2_ref[...] = w2_ref[...].astype(bf16)


def _cast_weights(wi, wsi, wso, wo, w1, w2):
    E, D, S = wi.shape
    H = wsi.shape[2]
    F = w1.shape[1]
    bf16 = jnp.bfloat16
    n = E
    spec3 = lambda d1, d2: pl.BlockSpec((1, d1, d2), lambda i: (i, 0, 0))
    return pl.pallas_call(
        _cast_kernel,
        grid=(n,),
        in_specs=[
            spec3(D, S), spec3(D, H), spec3(H, 4 * S), spec3(S, D),
            pl.BlockSpec((D // n, F), lambda i: (i, 0)),
            pl.BlockSpec((F // n, D), lambda i: (i, 0)),
        ],
        out_specs=[
            spec3(D, S), spec3(D, H), spec3(H, 4 * S), spec3(S, D),
            pl.BlockSpec((D // n, F), lambda i: (i, 0)),
            pl.BlockSpec((F // n, D), lambda i: (i, 0)),
        ],
        out_shape=[
            jax.ShapeDtypeStruct((E, D, S), bf16),
            jax.ShapeDtypeStruct((E, D, H), bf16),
            jax.ShapeDtypeStruct((E, H, 4 * S), bf16),
            jax.ShapeDtypeStruct((E, S, D), bf16),
            jax.ShapeDtypeStruct((D, F), bf16),
            jax.ShapeDtypeStruct((F, D), bf16),
        ],
    )(wi, wsi, wso, wo, w1, w2)


def kernel(x, x0, token_ids, W_in, W_sel_in, W_sel_out, W_out, d_param,
           resid_mix, ssm_scale, mlp_scale, W_mlp1, W_mlp2):
    B, T, D = x.shape
    E, _, S = W_in.shape

    x2 = x.reshape(T, D)
    x02 = x0.reshape(T, D)
    tid = token_ids.reshape(T)

    wi, wsi, wso, wo, w1, w2 = _cast_weights(
        W_in, W_sel_in, W_sel_out, W_out, W_mlp1, W_mlp2)
    pos, sidx, offs, xs, xs0 = _make_route_gather(E, T, D)(
        x2, x02, tid)
    ys = _run_ssm(xs, xs0, offs, resid_mix,
                  wi, wsi, wso, wo, d_param,
                  ssm_scale.reshape(1, D), mlp_scale.reshape(1, D),
                  w1, w2)
    out = _make_gather1(T, D)(ys, pos)
    return out.reshape(B, T, D)


# speedup vs baseline: 1.0418x; 1.0418x over previous
import functools

import jax
import jax.numpy as jnp
from jax import lax
from jax.experimental import pallas as pl
from jax.experimental.pallas import tpu as pltpu
from jax.experimental.pallas import tpu_sc as plsc


def _route_gather_body(E, T, b_per_w,
                       x_hbm, x0_hbm, tid_hbm,
                       pos_hbm, sidx_hbm, offs_hbm,
                       xs_hbm, xs0_hbm,
                       tid_v, r_v, pos_v, sidx_v, offs_v,
                       shared_sidx, idx_v, rows_v, rows0_v, sem, sem0):
    cid = lax.axis_index("c")
    sid = lax.axis_index("s")

    @pl.when(sid == 0)
    def _():
        pltpu.sync_copy(tid_hbm, tid_v)
        iota16 = lax.iota(jnp.int32, 16)
        nchunks = T // 16

        def hash_count(i, counts):
            t = tid_v[pl.ds(i * 16, 16)]
            h = t.astype(jnp.uint32)
            h = h ^ (h >> 16)
            h = h * jnp.uint32(2246822507)
            h = h ^ (h >> 13)
            h = h * jnp.uint32(3266489909)
            h = h ^ (h >> 16)
            r = (h % jnp.uint32(E)).astype(jnp.int32)
            r_v[pl.ds(i * 16, 16)] = r
            for e in range(E):
                c = jnp.sum((r == e).astype(jnp.int32))
                counts = counts + jnp.where(iota16 == e, c, 0)
            return counts

        counts = lax.fori_loop(0, nchunks, hash_count,
                               jnp.zeros((16,), jnp.int32))
        offs = plsc.cumsum(counts) - counts
        offs_v[pl.ds(0, 16)] = offs

        def place(i, running):
            r = r_v[pl.ds(i * 16, 16)]
            base = jnp.zeros((16,), jnp.int32)
            newrun = running
            for e in range(E):
                m = r == e
                mi = m.astype(jnp.int32)
                pre = plsc.cumsum(mi)
                run_e = jnp.sum(jnp.where(iota16 == e, running, 0))
                base = jnp.where(m, run_e + pre - 1, base)
                newrun = newrun + jnp.where(iota16 == e, jnp.sum(mi), 0)
            pos_v[pl.ds(i * 16, 16)] = base
            tok = lax.iota(jnp.int32, 16) + i * 16
            plsc.store_scatter(sidx_v, [base], tok)
            return newrun

        lax.fori_loop(0, nchunks, place, offs)
        pltpu.sync_copy(sidx_v, shared_sidx)

    @pl.when((sid == 0) & (cid == 0))
    def _():
        pltpu.sync_copy(pos_v, pos_hbm)
        pltpu.sync_copy(sidx_v, sidx_hbm)
        pltpu.sync_copy(offs_v, offs_hbm)

    plsc.subcore_barrier()

    wid = sid * 2 + cid
    base = wid * b_per_w
    pltpu.sync_copy(shared_sidx.at[pl.ds(base, b_per_w)], idx_v)
    cp = pltpu.async_copy(x_hbm.at[idx_v], rows_v, sem)
    cp0 = pltpu.async_copy(x0_hbm.at[idx_v], rows0_v, sem0)
    cp.wait()
    cp0.wait()
    pltpu.sync_copy(rows_v, xs_hbm.at[pl.ds(base, b_per_w)])
    pltpu.sync_copy(rows0_v, xs0_hbm.at[pl.ds(base, b_per_w)])


def _make_route_gather(E, T, D):
    b_per_w = T // 32
    mesh = plsc.VectorSubcoreMesh(core_axis_name="c", subcore_axis_name="s")
    i32 = jnp.int32
    f32 = jnp.float32
    return functools.partial(
        pl.kernel,
        out_type=(jax.ShapeDtypeStruct((T,), i32),
                  jax.ShapeDtypeStruct((T,), i32),
                  jax.ShapeDtypeStruct((16,), i32),
                  jax.ShapeDtypeStruct((T, D), f32),
                  jax.ShapeDtypeStruct((T, D), f32)),
        mesh=mesh,
        scratch_types=[pltpu.VMEM((T,), i32),
                       pltpu.VMEM((T,), i32),
                       pltpu.VMEM((T,), i32),
                       pltpu.VMEM((T,), i32),
                       pltpu.VMEM((16,), i32),
                       pltpu.VMEM_SHARED((T,), i32),
                       pltpu.VMEM((b_per_w,), i32),
                       pltpu.VMEM((b_per_w, D), f32),
                       pltpu.VMEM((b_per_w, D), f32),
                       pltpu.SemaphoreType.DMA,
                       pltpu.SemaphoreType.DMA],
        compiler_params=pltpu.CompilerParams(needs_layout_passes=False),
    )(functools.partial(_route_gather_body, E, T, b_per_w))


def _gather1_body(b_per_w, x_hbm, idx_hbm, xs_hbm, idx_v, rows_v, sem):
    wid = lax.axis_index("s") * 2 + lax.axis_index("c")
    base = wid * b_per_w
    pltpu.sync_copy(idx_hbm.at[pl.ds(base, b_per_w)], idx_v)
    pltpu.async_copy(x_hbm.at[idx_v], rows_v, sem).wait()
    pltpu.sync_copy(rows_v, xs_hbm.at[pl.ds(base, b_per_w)])


def _make_gather1(T, D):
    b_per_w = T // 32
    mesh = plsc.VectorSubcoreMesh(core_axis_name="c", subcore_axis_name="s")
    return functools.partial(
        pl.kernel,
        out_type=jax.ShapeDtypeStruct((T, D), jnp.float32),
        mesh=mesh,
        scratch_types=[pltpu.VMEM((b_per_w,), jnp.int32),
                       pltpu.VMEM((b_per_w, D), jnp.float32),
                       pltpu.SemaphoreType.DMA],
    )(functools.partial(_gather1_body, b_per_w))


_BLK = 256


def _ssm_kernel(xs_ref, xs0_ref, offs_ref, rm_ref,
                wis_ref, wso_ref, wo_ref, dp_ref, ssc_ref, msc_ref,
                w1_ref, w2_ref, ys_ref, hcar_ref):
    f32 = jnp.float32
    BLK = _BLK
    S = wo_ref.shape[1]
    D = wo_ref.shape[2]
    E = wis_ref.shape[0]
    pid = pl.program_id(0)

    @pl.when(pid == 0)
    def _():
        hcar_ref[...] = jnp.zeros_like(hcar_ref)

    xm = rm_ref[0:1, :] * xs_ref[...] + rm_ref[1:2, :] * xs0_ref[...]
    xn = xm * lax.rsqrt(jnp.mean(xm * xm, axis=-1, keepdims=True) + 1e-6)
    xnb = xn.astype(jnp.bfloat16)

    t0 = pid * BLK
    t_hi = t0 + BLK - 1
    ti = lax.broadcasted_iota(jnp.int32, (BLK, 1), 0) + t0
    e_lo = jnp.int32(0)
    e_hi = jnp.int32(0)
    for e in range(1, E):
        off_e = offs_ref[0, e]
        e_lo = jnp.where(off_e <= t0, jnp.int32(e), e_lo)
        e_hi = jnp.where(off_e <= t_hi, jnp.int32(e), e_hi)

    def _mask(e):
        lo = offs_ref[0, e]
        hi = offs_ref[0, e + 1]
        return ((ti >= lo) & (ti < hi)).astype(f32)

    SH = wis_ref.shape[2]

    def body1(e, USL):
        m = _mask(e)
        wis = wis_ref[pl.ds(e, 1)][0]
        return USL + m * jnp.dot(xnb, wis, preferred_element_type=f32)

    USL = lax.fori_loop(e_lo, e_hi + 1, body1, jnp.zeros((BLK, SH), f32))
    U = USL[:, 0:S]
    SELR = USL[:, S:]
    selb = (SELR * jax.nn.sigmoid(SELR)).astype(jnp.bfloat16)

    def body2(e, carry):
        SO, DP = carry
        m = _mask(e)
        wso = wso_ref[pl.ds(e, 1)][0]
        SO = SO + m * jnp.dot(selb, wso, preferred_element_type=f32)
        DP = DP + m * dp_ref[pl.ds(e, 1), :]
        return SO, DP

    SO, DP = lax.fori_loop(e_lo, e_hi + 1, body2,
                           (jnp.zeros((BLK, 4 * S), f32),
                            jnp.zeros((BLK, S), f32)))

    a = jax.nn.sigmoid(SO[:, 0:S])
    b = jnp.tanh(SO[:, S:2 * S])
    c = jnp.tanh(SO[:, 2 * S:3 * S])
    dg = jax.nn.sigmoid(SO[:, 3 * S:4 * S])

    amult = jnp.ones((BLK, 1), f32)
    for e in range(E):
        amult = jnp.where(ti == offs_ref[0, e], 0.0, amult)
    A = a * amult
    Hs = b * U
    ri = lax.broadcasted_iota(jnp.int32, (BLK, 1), 0)
    d = 1
    while d < BLK:
        h_sh = jnp.where(ri >= d, pltpu.roll(Hs, d, axis=0), 0.0)
        a_sh = jnp.where(ri >= d, pltpu.roll(A, d, axis=0), 1.0)
        Hs = Hs + A * h_sh
        A = A * a_sh
        d *= 2
    h = Hs + A * hcar_ref[0:1, :]
    hcar_ref[0:1, :] = h[BLK - 1:BLK, :]

    Y = c * h + DP * dg * U
    yb = Y.astype(jnp.bfloat16)

    def body3(e, OUT):
        m = _mask(e)
        wo = wo_ref[pl.ds(e, 1)][0]
        return OUT + m * jnp.dot(yb, wo, preferred_element_type=f32)

    ssm_out = lax.fori_loop(e_lo, e_hi + 1, body3, jnp.zeros((BLK, D), f32))

    xm2 = xm + ssc_ref[...] * ssm_out
    xn2 = xm2 * lax.rsqrt(jnp.mean(xm2 * xm2, axis=-1, keepdims=True) + 1e-6)
    hmid = jnp.dot(xn2.astype(jnp.bfloat16), w1_ref[...],
                   preferred_element_type=f32)
    hact = jnp.maximum(hmid, 0.0)
    hact = hact * hact
    mlp = jnp.dot(hact.astype(jnp.bfloat16), w2_ref[...],
                  preferred_element_type=f32)
    ys_ref[...] = xm2 + msc_ref[...] * mlp


def _run_ssm(xs, xs0, offs, resid_mix, wis, wso, wo, dp,
             ssc, msc, w1, w2):
    T, D = xs.shape
    E, S, _ = wo.shape
    SH = wis.shape[2]
    H = SH - S
    F = w1.shape[1]
    nblk = T // _BLK
    full = lambda *shape: pl.BlockSpec(shape, lambda i: (0,) * len(shape))
    return pl.pallas_call(
        _ssm_kernel,
        grid=(nblk,),
        in_specs=[
            pl.BlockSpec((_BLK, D), lambda i: (i, 0)),
            pl.BlockSpec((_BLK, D), lambda i: (i, 0)),
            pl.BlockSpec((1, 16), lambda i: (0, 0),
                         memory_space=pltpu.SMEM),
            full(2, D),
            full(E, D, SH),
            full(E, H, 4 * S),
            full(E, S, D),
            full(E, S),
            full(1, D),
            full(1, D),
            full(D, F),
            full(F, D),
        ],
        out_specs=pl.BlockSpec((_BLK, D), lambda i: (i, 0)),
        out_shape=jax.ShapeDtypeStruct((T, D), jnp.float32),
        scratch_shapes=[pltpu.VMEM((8, S), jnp.float32)],
    )(xs, xs0, offs.reshape(1, 16), resid_mix, wis, wso, wo, dp,
      ssc, msc, w1, w2)


def _cast_kernel(wi_ref, wsi_ref, wso_ref, wo_ref, w1_ref, w2_ref,
                 owis_ref, owso_ref, owo_ref, ow1_ref, ow2_ref):
    bf16 = jnp.bfloat16
    S = wi_ref.shape[2]
    owis_ref[:, :, 0:S] = wi_ref[...].astype(bf16)
    owis_ref[:, :, S:] = wsi_ref[...].astype(bf16)
    owso_ref[...] = wso_ref[...].astype(bf16)
    owo_ref[...] = wo_ref[...].astype(bf16)
    ow1_ref[...] = w1_ref[...].astype(bf16)
    ow2_ref[...] = w2_ref[...].astype(bf16)


def _cast_weights(wi, wsi, wso, wo, w1, w2):
    E, D, S = wi.shape
    H = wsi.shape[2]
    F = w1.shape[1]
    bf16 = jnp.bfloat16
    n = E
    spec3 = lambda d1, d2: pl.BlockSpec((1, d1, d2), lambda i: (i, 0, 0))
    return pl.pallas_call(
        _cast_kernel,
        grid=(n,),
        in_specs=[
            spec3(D, S), spec3(D, H), spec3(H, 4 * S), spec3(S, D),
            pl.BlockSpec((D // n, F), lambda i: (i, 0)),
            pl.BlockSpec((F // n, D), lambda i: (i, 0)),
        ],
        out_specs=[
            spec3(D, S + H), spec3(H, 4 * S), spec3(S, D),
            pl.BlockSpec((D // n, F), lambda i: (i, 0)),
            pl.BlockSpec((F // n, D), lambda i: (i, 0)),
        ],
        out_shape=[
            jax.ShapeDtypeStruct((E, D, S + H), bf16),
            jax.ShapeDtypeStruct((E, H, 4 * S), bf16),
            jax.ShapeDtypeStruct((E, S, D), bf16),
            jax.ShapeDtypeStruct((D, F), bf16),
            jax.ShapeDtypeStruct((F, D), bf16),
        ],
    )(wi, wsi, wso, wo, w1, w2)


def kernel(x, x0, token_ids, W_in, W_sel_in, W_sel_out, W_out, d_param,
           resid_mix, ssm_scale, mlp_scale, W_mlp1, W_mlp2):
    B, T, D = x.shape
    E, _, S = W_in.shape

    x2 = x.reshape(T, D)
    x02 = x0.reshape(T, D)
    tid = token_ids.reshape(T)

    wis, wso, wo, w1, w2 = _cast_weights(
        W_in, W_sel_in, W_sel_out, W_out, W_mlp1, W_mlp2)
    pos, sidx, offs, xs, xs0 = _make_route_gather(E, T, D)(
        x2, x02, tid)
    ys = _run_ssm(xs, xs0, offs, resid_mix,
                  wis, wso, wo, d_param,
                  ssm_scale.reshape(1, D), mlp_scale.reshape(1, D),
                  w1, w2)
    out = _make_gather1(T, D)(ys, pos)
    return out.reshape(B, T, D)


# speedup vs baseline: 1.0909x; 1.0471x over previous
import functools

import jax
import jax.numpy as jnp
from jax import lax
from jax.experimental import pallas as pl
from jax.experimental.pallas import tpu as pltpu
from jax.experimental.pallas import tpu_sc as plsc


def _route_gather_body(E, T, b_per_w,
                       x_hbm, x0_hbm, tid_hbm,
                       pos_hbm, sidx_hbm, offs_hbm,
                       xs_hbm, xs0_hbm,
                       tid_v, r_v, rank_v, pos_v, sidx_v, offs_v, run_v,
                       shared_sidx, idx_v, rows_v, rows0_v, sem, sem0):
    cid = lax.axis_index("c")
    sid = lax.axis_index("s")

    @pl.when(sid == 0)
    def _():
        pltpu.sync_copy(tid_hbm, tid_v)
        nchunks = T // 16
        run_v[pl.ds(0, 16)] = jnp.zeros((16,), jnp.int32)

        def hash_count(i, _):
            t = tid_v[pl.ds(i * 16, 16)]
            h = t.astype(jnp.uint32)
            h = h ^ (h >> 16)
            h = h * jnp.uint32(2246822507)
            h = h ^ (h >> 13)
            h = h * jnp.uint32(3266489909)
            h = h ^ (h >> 16)
            r = (h % jnp.uint32(E)).astype(jnp.int32)
            r_v[pl.ds(i * 16, 16)] = r
            rank, last = plsc.scan_count(r)
            rank_v[pl.ds(i * 16, 16)] = rank
            cur = plsc.load_gather(run_v, [r])
            plsc.store_scatter(run_v, [r], cur + rank, mask=last)
            return 0

        lax.fori_loop(0, nchunks, hash_count, 0)
        counts = run_v[pl.ds(0, 16)]
        offs = plsc.cumsum(counts) - counts
        offs_v[pl.ds(0, 16)] = offs
        run_v[pl.ds(0, 16)] = offs

        def place(i, _):
            r = r_v[pl.ds(i * 16, 16)]
            rank = rank_v[pl.ds(i * 16, 16)]
            base = plsc.load_gather(run_v, [r])
            posv = base + rank - 1
            pos_v[pl.ds(i * 16, 16)] = posv
            tok = lax.iota(jnp.int32, 16) + i * 16
            plsc.store_scatter(sidx_v, [posv], tok)
            _, last = plsc.scan_count(r)
            plsc.store_scatter(run_v, [r], posv + 1, mask=last)
            return 0

        lax.fori_loop(0, nchunks, place, 0)
        pltpu.sync_copy(sidx_v, shared_sidx)

    @pl.when((sid == 0) & (cid == 0))
    def _():
        pltpu.sync_copy(pos_v, pos_hbm)
        pltpu.sync_copy(sidx_v, sidx_hbm)
        pltpu.sync_copy(offs_v, offs_hbm)

    plsc.subcore_barrier()

    wid = sid * 2 + cid
    base = wid * b_per_w
    pltpu.sync_copy(shared_sidx.at[pl.ds(base, b_per_w)], idx_v)
    cp = pltpu.async_copy(x_hbm.at[idx_v], rows_v, sem)
    cp0 = pltpu.async_copy(x0_hbm.at[idx_v], rows0_v, sem0)
    cp.wait()
    cp0.wait()
    pltpu.sync_copy(rows_v, xs_hbm.at[pl.ds(base, b_per_w)])
    pltpu.sync_copy(rows0_v, xs0_hbm.at[pl.ds(base, b_per_w)])


def _make_route_gather(E, T, D):
    b_per_w = T // 32
    mesh = plsc.VectorSubcoreMesh(core_axis_name="c", subcore_axis_name="s")
    i32 = jnp.int32
    f32 = jnp.float32
    return functools.partial(
        pl.kernel,
        out_type=(jax.ShapeDtypeStruct((T,), i32),
                  jax.ShapeDtypeStruct((T,), i32),
                  jax.ShapeDtypeStruct((16,), i32),
                  jax.ShapeDtypeStruct((T, D), f32),
                  jax.ShapeDtypeStruct((T, D), f32)),
        mesh=mesh,
        scratch_types=[pltpu.VMEM((T,), i32),
                       pltpu.VMEM((T,), i32),
                       pltpu.VMEM((T,), i32),
                       pltpu.VMEM((T,), i32),
                       pltpu.VMEM((T,), i32),
                       pltpu.VMEM((16,), i32),
                       pltpu.VMEM((16,), i32),
                       pltpu.VMEM_SHARED((T,), i32),
                       pltpu.VMEM((b_per_w,), i32),
                       pltpu.VMEM((b_per_w, D), f32),
                       pltpu.VMEM((b_per_w, D), f32),
                       pltpu.SemaphoreType.DMA,
                       pltpu.SemaphoreType.DMA],
        compiler_params=pltpu.CompilerParams(needs_layout_passes=False),
    )(functools.partial(_route_gather_body, E, T, b_per_w))


def _gather1_body(b_per_w, x_hbm, idx_hbm, xs_hbm, idx_v, rows_v, sem):
    wid = lax.axis_index("s") * 2 + lax.axis_index("c")
    base = wid * b_per_w
    pltpu.sync_copy(idx_hbm.at[pl.ds(base, b_per_w)], idx_v)
    pltpu.async_copy(x_hbm.at[idx_v], rows_v, sem).wait()
    pltpu.sync_copy(rows_v, xs_hbm.at[pl.ds(base, b_per_w)])


def _make_gather1(T, D):
    b_per_w = T // 32
    mesh = plsc.VectorSubcoreMesh(core_axis_name="c", subcore_axis_name="s")
    return functools.partial(
        pl.kernel,
        out_type=jax.ShapeDtypeStruct((T, D), jnp.float32),
        mesh=mesh,
        scratch_types=[pltpu.VMEM((b_per_w,), jnp.int32),
                       pltpu.VMEM((b_per_w, D), jnp.float32),
                       pltpu.SemaphoreType.DMA],
    )(functools.partial(_gather1_body, b_per_w))


_BLK = 256


def _ssm_kernel(xs_ref, xs0_ref, offs_ref, rm_ref,
                wis_ref, wso_ref, wo_ref, dp_ref, ssc_ref, msc_ref,
                w1_ref, w2_ref, ys_ref, hcar_ref):
    f32 = jnp.float32
    BLK = _BLK
    S = wo_ref.shape[1]
    D = wo_ref.shape[2]
    E = wis_ref.shape[0]
    pid = pl.program_id(0)

    @pl.when(pid == 0)
    def _():
        hcar_ref[...] = jnp.zeros_like(hcar_ref)

    xm = rm_ref[0:1, :] * xs_ref[...] + rm_ref[1:2, :] * xs0_ref[...]
    xn = xm * lax.rsqrt(jnp.mean(xm * xm, axis=-1, keepdims=True) + 1e-6)
    xnb = xn.astype(jnp.bfloat16)

    t0 = pid * BLK
    t_hi = t0 + BLK - 1
    ti = lax.broadcasted_iota(jnp.int32, (BLK, 1), 0) + t0
    e_lo = jnp.int32(0)
    e_hi = jnp.int32(0)
    for e in range(1, E):
        off_e = offs_ref[0, e]
        e_lo = jnp.where(off_e <= t0, jnp.int32(e), e_lo)
        e_hi = jnp.where(off_e <= t_hi, jnp.int32(e), e_hi)

    def _mask(e):
        lo = offs_ref[0, e]
        hi = offs_ref[0, e + 1]
        return ((ti >= lo) & (ti < hi)).astype(f32)

    SH = wis_ref.shape[2]

    def body1(e, USL):
        m = _mask(e)
        wis = wis_ref[pl.ds(e, 1)][0]
        return USL + m * jnp.dot(xnb, wis, preferred_element_type=f32)

    USL = lax.fori_loop(e_lo, e_hi + 1, body1, jnp.zeros((BLK, SH), f32))
    U = USL[:, 0:S]
    SELR = USL[:, S:]
    selb = (SELR * jax.nn.sigmoid(SELR)).astype(jnp.bfloat16)

    def body2(e, carry):
        SO, DP = carry
        m = _mask(e)
        wso = wso_ref[pl.ds(e, 1)][0]
        SO = SO + m * jnp.dot(selb, wso, preferred_element_type=f32)
        DP = DP + m * dp_ref[pl.ds(e, 1), :]
        return SO, DP

    SO, DP = lax.fori_loop(e_lo, e_hi + 1, body2,
                           (jnp.zeros((BLK, 4 * S), f32),
                            jnp.zeros((BLK, S), f32)))

    a = jax.nn.sigmoid(SO[:, 0:S])
    b = jnp.tanh(SO[:, S:2 * S])
    c = jnp.tanh(SO[:, 2 * S:3 * S])
    dg = jax.nn.sigmoid(SO[:, 3 * S:4 * S])

    amult = jnp.ones((BLK, 1), f32)
    for e in range(E):
        amult = jnp.where(ti == offs_ref[0, e], 0.0, amult)
    A = a * amult
    Hs = b * U
    ri = lax.broadcasted_iota(jnp.int32, (BLK, 1), 0)
    d = 1
    while d < BLK:
        h_sh = jnp.where(ri >= d, pltpu.roll(Hs, d, axis=0), 0.0)
        a_sh = jnp.where(ri >= d, pltpu.roll(A, d, axis=0), 1.0)
        Hs = Hs + A * h_sh
        A = A * a_sh
        d *= 2
    h = Hs + A * hcar_ref[0:1, :]
    hcar_ref[0:1, :] = h[BLK - 1:BLK, :]

    Y = c * h + DP * dg * U
    yb = Y.astype(jnp.bfloat16)

    def body3(e, OUT):
        m = _mask(e)
        wo = wo_ref[pl.ds(e, 1)][0]
        return OUT + m * jnp.dot(yb, wo, preferred_element_type=f32)

    ssm_out = lax.fori_loop(e_lo, e_hi + 1, body3, jnp.zeros((BLK, D), f32))

    xm2 = xm + ssc_ref[...] * ssm_out
    xn2 = xm2 * lax.rsqrt(jnp.mean(xm2 * xm2, axis=-1, keepdims=True) + 1e-6)
    hmid = jnp.dot(xn2.astype(jnp.bfloat16), w1_ref[...],
                   preferred_element_type=f32)
    hact = jnp.maximum(hmid, 0.0)
    hact = hact * hact
    mlp = jnp.dot(hact.astype(jnp.bfloat16), w2_ref[...],
                  preferred_element_type=f32)
    ys_ref[...] = xm2 + msc_ref[...] * mlp


def _run_ssm(xs, xs0, offs, resid_mix, wis, wso, wo, dp,
             ssc, msc, w1, w2):
    T, D = xs.shape
    E, S, _ = wo.shape
    SH = wis.shape[2]
    H = SH - S
    F = w1.shape[1]
    nblk = T // _BLK
    full = lambda *shape: pl.BlockSpec(shape, lambda i: (0,) * len(shape))
    return pl.pallas_call(
        _ssm_kernel,
        grid=(nblk,),
        in_specs=[
            pl.BlockSpec((_BLK, D), lambda i: (i, 0)),
            pl.BlockSpec((_BLK, D), lambda i: (i, 0)),
            pl.BlockSpec((1, 16), lambda i: (0, 0),
                         memory_space=pltpu.SMEM),
            full(2, D),
            full(E, D, SH),
            full(E, H, 4 * S),
            full(E, S, D),
            full(E, S),
            full(1, D),
            full(1, D),
            full(D, F),
            full(F, D),
        ],
        out_specs=pl.BlockSpec((_BLK, D), lambda i: (i, 0)),
        out_shape=jax.ShapeDtypeStruct((T, D), jnp.float32),
        scratch_shapes=[pltpu.VMEM((8, S), jnp.float32)],
    )(xs, xs0, offs.reshape(1, 16), resid_mix, wis, wso, wo, dp,
      ssc, msc, w1, w2)


def _cast_kernel(wi_ref, wsi_ref, wso_ref, wo_ref, w1_ref, w2_ref,
                 owis_ref, owso_ref, owo_ref, ow1_ref, ow2_ref):
    bf16 = jnp.bfloat16
    S = wi_ref.shape[2]
    owis_ref[:, :, 0:S] = wi_ref[...].astype(bf16)
    owis_ref[:, :, S:] = wsi_ref[...].astype(bf16)
    owso_ref[...] = wso_ref[...].astype(bf16)
    owo_ref[...] = wo_ref[...].astype(bf16)
    ow1_ref[...] = w1_ref[...].astype(bf16)
    ow2_ref[...] = w2_ref[...].astype(bf16)


def _cast_weights(wi, wsi, wso, wo, w1, w2):
    E, D, S = wi.shape
    H = wsi.shape[2]
    F = w1.shape[1]
    bf16 = jnp.bfloat16
    n = E
    spec3 = lambda d1, d2: pl.BlockSpec((1, d1, d2), lambda i: (i, 0, 0))
    return pl.pallas_call(
        _cast_kernel,
        grid=(n,),
        in_specs=[
            spec3(D, S), spec3(D, H), spec3(H, 4 * S), spec3(S, D),
            pl.BlockSpec((D // n, F), lambda i: (i, 0)),
            pl.BlockSpec((F // n, D), lambda i: (i, 0)),
        ],
        out_specs=[
            spec3(D, S + H), spec3(H, 4 * S), spec3(S, D),
            pl.BlockSpec((D // n, F), lambda i: (i, 0)),
            pl.BlockSpec((F // n, D), lambda i: (i, 0)),
        ],
        out_shape=[
            jax.ShapeDtypeStruct((E, D, S + H), bf16),
            jax.ShapeDtypeStruct((E, H, 4 * S), bf16),
            jax.ShapeDtypeStruct((E, S, D), bf16),
            jax.ShapeDtypeStruct((D, F), bf16),
            jax.ShapeDtypeStruct((F, D), bf16),
        ],
    )(wi, wsi, wso, wo, w1, w2)


def kernel(x, x0, token_ids, W_in, W_sel_in, W_sel_out, W_out, d_param,
           resid_mix, ssm_scale, mlp_scale, W_mlp1, W_mlp2):
    B, T, D = x.shape
    E, _, S = W_in.shape

    x2 = x.reshape(T, D)
    x02 = x0.reshape(T, D)
    tid = token_ids.reshape(T)

    wis, wso, wo, w1, w2 = _cast_weights(
        W_in, W_sel_in, W_sel_out, W_out, W_mlp1, W_mlp2)
    pos, sidx, offs, xs, xs0 = _make_route_gather(E, T, D)(
        x2, x02, tid)
    ys = _run_ssm(xs, xs0, offs, resid_mix,
                  wis, wso, wo, d_param,
                  ssm_scale.reshape(1, D), mlp_scale.reshape(1, D),
                  w1, w2)
    out = _make_gather1(T, D)(ys, pos)
    return out.reshape(B, T, D)
